# baseline (speedup 1.0000x reference)
import jax
import jax.numpy as jnp
from jax.experimental import pallas as pl
from jax.experimental.pallas import tpu as pltpu

_SIZE_REG = 1.0
_ENT_REG = 1.0
_EPS = 1e-15


def _stream_kernel(m_ref, f_ref, c_ref, s_ref, e_ref):
    m = m_ref[...]
    f_ref[...] = m
    c_ref[...] = 1.0 - m
    ent = -m * jnp.log(m + _EPS) - (1.0 - m) * jnp.log(1.0 - m + _EPS)
    s_ref[...] = jnp.sum(m).reshape(1, 1, 1)
    e_ref[...] = jnp.sum(ent).reshape(1, 1, 1)


def kernel(edge_filter, mask):
    B, E = edge_filter.shape
    n = B * E
    m2 = mask.reshape(B, E)
    RB = 128
    G = B // RB
    f, c, s, e = pl.pallas_call(
        _stream_kernel,
        grid=(G,),
        in_specs=[pl.BlockSpec((RB, E), lambda i: (i, 0))],
        out_specs=[
            pl.BlockSpec((RB, E), lambda i: (i, 0)),
            pl.BlockSpec((RB, E), lambda i: (i, 0)),
            pl.BlockSpec((1, 1, 1), lambda i: (i, 0, 0)),
            pl.BlockSpec((1, 1, 1), lambda i: (i, 0, 0)),
        ],
        out_shape=[
            jax.ShapeDtypeStruct((B, E), mask.dtype),
            jax.ShapeDtypeStruct((B, E), mask.dtype),
            jax.ShapeDtypeStruct((G, 1, 1), jnp.float32),
            jax.ShapeDtypeStruct((G, 1, 1), jnp.float32),
        ],
        compiler_params=pltpu.CompilerParams(
            dimension_semantics=("parallel",),
        ),
    )(m2)
    inv_n = 1.0 / n
    size_loss = jnp.sum(s) * (_SIZE_REG * inv_n)
    ent_loss = jnp.sum(e) * (_ENT_REG * inv_n)
    return f, c, size_loss, ent_loss


# speedup vs baseline: 1.5252x; 1.5252x over previous
import jax
import jax.numpy as jnp
from jax.experimental import pallas as pl
from jax.experimental.pallas import tpu as pltpu

_SIZE_REG = 1.0
_ENT_REG = 1.0
_EPS = 1e-15


def _stream_kernel(m_ref, f_ref, c_ref, s_ref, e_ref):
    m = m_ref[...].reshape(f_ref.shape)
    f_ref[...] = m
    c_ref[...] = 1.0 - m
    ent = -m * jnp.log(m + _EPS) - (1.0 - m) * jnp.log(1.0 - m + _EPS)
    s_ref[...] = jnp.sum(m).reshape(1, 1, 1)
    e_ref[...] = jnp.sum(ent).reshape(1, 1, 1)


def kernel(edge_filter, mask):
    B, E = edge_filter.shape
    n = B * E
    RB = 128
    G = B // RB
    f, c, s, e = pl.pallas_call(
        _stream_kernel,
        grid=(G,),
        in_specs=[pl.BlockSpec((RB * E,), lambda i: (i,))],
        out_specs=[
            pl.BlockSpec((RB, E), lambda i: (i, 0)),
            pl.BlockSpec((RB, E), lambda i: (i, 0)),
            pl.BlockSpec((1, 1, 1), lambda i: (i, 0, 0)),
            pl.BlockSpec((1, 1, 1), lambda i: (i, 0, 0)),
        ],
        out_shape=[
            jax.ShapeDtypeStruct((B, E), mask.dtype),
            jax.ShapeDtypeStruct((B, E), mask.dtype),
            jax.ShapeDtypeStruct((G, 1, 1), jnp.float32),
            jax.ShapeDtypeStruct((G, 1, 1), jnp.float32),
        ],
        compiler_params=pltpu.CompilerParams(
            dimension_semantics=("parallel",),
        ),
    )(mask)
    inv_n = 1.0 / n
    size_loss = jnp.sum(s) * (_SIZE_REG * inv_n)
    ent_loss = jnp.sum(e) * (_ENT_REG * inv_n)
    return f, c, size_loss, ent_loss
